# baseline (speedup 1.0000x reference)
import functools

import jax
import jax.numpy as jnp
from jax import lax
from jax.experimental import pallas as pl
from jax.experimental.pallas import tpu as pltpu
from jax.experimental.pallas import tpu_sc as plsc

VOCAB = 1000000
DIM = 64
BATCH = 16384
HIST = 50

_NW = 32
_BPW = BATCH // _NW
_CB = 8
_NCHUNK = _BPW // _CB
_HALF = _BPW // 2
_HP = 56


def _body(xt_hbm, table_hbm, out_hbm, idx_ht, idx_bv, buf0, buf1, gsem0, gsem1):
    wid = lax.axis_index("s") * 2 + lax.axis_index("c")
    b0 = wid * _BPW

    iota = lax.iota(jnp.int32, 16)
    for half in range(2):
        pltpu.sync_copy(xt_hbm.at[:, pl.ds(b0 + half * _HALF, _HALF)], idx_ht)
        for bbg in range(_HALF // 16):
            base = (half * _HALF + bbg * 16) * _HP + iota * _HP

            @pl.loop(0, HIST)
            def _(h):
                v = idx_ht[h, pl.ds(bbg * 16, 16)]
                plsc.store_scatter(idx_bv, [base + h], v)

    bufs = (buf0, buf1)
    gsems = (gsem0, gsem1)

    def fire(c, b):
        for k in range(_CB):
            off = pl.multiple_of((c * _CB + k) * _HP, 8)
            pltpu.async_copy(
                table_hbm.at[idx_bv.at[pl.ds(off, HIST)]],
                bufs[b].at[k],
                gsems[b],
            )

    def drain(b):
        for k in range(_CB):
            pltpu.make_async_copy(
                table_hbm.at[idx_bv.at[pl.ds(k * _HP, HIST)]],
                bufs[b].at[k],
                gsems[b],
            ).wait()

    fire(0, 0)

    @pl.loop(0, _NCHUNK, step=2)
    def _(c):
        for b in range(2):
            cc = c + b

            @pl.when(cc + 1 < _NCHUNK)
            def _():
                fire(cc + 1, 1 - b)

            drain(b)
            pltpu.sync_copy(bufs[b], out_hbm.at[pl.ds(b0 + cc * _CB, _CB)])


@jax.jit
def kernel(x, table):
    mesh = plsc.VectorSubcoreMesh(core_axis_name="c", subcore_axis_name="s")
    out = pl.kernel(
        _body,
        out_type=jax.ShapeDtypeStruct((BATCH, HIST, DIM), jnp.float32),
        mesh=mesh,
        scratch_types=[
            pltpu.VMEM((HIST, _HALF), jnp.int32),
            pltpu.VMEM((_BPW * _HP,), jnp.int32),
            pltpu.VMEM((_CB, HIST, DIM), jnp.float32),
            pltpu.VMEM((_CB, HIST, DIM), jnp.float32),
            pltpu.SemaphoreType.DMA,
            pltpu.SemaphoreType.DMA,
        ],
        compiler_params=pltpu.CompilerParams(
            use_tc_tiling_on_sc=False, needs_layout_passes=False
        ),
    )(x.T.astype(jnp.int32), table)
    return out
